# baseline (speedup 1.0000x reference)
import functools

import jax
import jax.numpy as jnp
from jax import lax
from jax.experimental import pallas as pl
from jax.experimental.pallas import tpu as pltpu
from jax.experimental.pallas import tpu_sc as plsc

_NC = 2
_NS = 16
_NW = _NC * _NS
_L = 16


@functools.lru_cache(maxsize=None)
def _make_sc_kernel(n_total: int, chunk: int):
    assert n_total % (_NW * chunk) == 0
    per_w = n_total // _NW
    n_chunks = per_w // chunk

    mesh = plsc.VectorSubcoreMesh(core_axis_name="c", subcore_axis_name="s")

    assert n_chunks % 2 == 0
    n_half = n_chunks // 2
    n_vec = chunk // _L

    @functools.partial(
        pl.kernel,
        out_type=jax.ShapeDtypeStruct((n_total,), jnp.float32),
        mesh=mesh,
        compiler_params=pltpu.CompilerParams(needs_layout_passes=False),
        scratch_types=[
            pltpu.VMEM((_L,), jnp.float32),
            pltpu.VMEM((_L,), jnp.float32),
            pltpu.VMEM((_L,), jnp.float32),
            pltpu.VMEM((_L,), jnp.float32),
            pltpu.VMEM((16 * _L,), jnp.float32),
            pltpu.VMEM((chunk,), jnp.float32),
            pltpu.VMEM((chunk,), jnp.float32),
            pltpu.VMEM((chunk,), jnp.float32),
            pltpu.VMEM((chunk,), jnp.float32),
            pltpu.SemaphoreType.DMA,
            pltpu.SemaphoreType.DMA,
            pltpu.SemaphoreType.DMA,
            pltpu.SemaphoreType.DMA,
        ],
    )
    def k(x_hbm, la_hbm, lb_hbm, mn_hbm, mx_hbm, out_hbm,
          la_v, lb_v, mn_v, mx_v, full_v,
          in_v0, in_v1, out_v0, out_v1,
          in_s0, in_s1, out_s0, out_s1):
        in_bufs, out_bufs = (in_v0, in_v1), (out_v0, out_v1)
        in_sems, out_sems = (in_s0, in_s1), (out_s0, out_s1)
        wid = lax.axis_index("s") * _NC + lax.axis_index("c")
        base = wid * per_w

        pltpu.sync_copy(la_hbm, la_v)
        pltpu.sync_copy(lb_hbm, lb_v)
        pltpu.sync_copy(mn_hbm, mn_v)
        pltpu.sync_copy(mx_hbm, mx_v)

        a_vec = la_v[...]
        b_vec = lb_v[...]
        mn = mn_v[...]
        mx = mx_v[...]
        row_ids = lax.iota(jnp.int32, _L) * _L
        for l in range(16):
            y = a_vec * (l / 16.0) + b_vec
            yf = y.astype(jnp.int32).astype(jnp.float32)
            plsc.store_scatter(full_v, [row_ids + l], jnp.clip(yf, mn, mx))

        def start_in(g, b):
            off = base + g * chunk
            pltpu.async_copy(x_hbm.at[pl.ds(off, chunk)], in_bufs[b], in_sems[b])

        def start_out(g, b):
            off = base + g * chunk
            pltpu.async_copy(out_bufs[b], out_hbm.at[pl.ds(off, chunk)], out_sems[b])

        def wait_in(g, b):
            off = base + g * chunk
            pltpu.make_async_copy(
                x_hbm.at[pl.ds(off, chunk)], in_bufs[b], in_sems[b]).wait()

        def wait_out(g, b):
            off = base + g * chunk
            pltpu.make_async_copy(
                out_bufs[b], out_hbm.at[pl.ds(off, chunk)], out_sems[b]).wait()

        start_in(0, 0)
        start_in(1, 1)

        def outer(g2, _):
            for b in range(2):
                g = g2 * 2 + b
                wait_in(g, b)
                pl.when(g2 >= 1)(lambda: wait_out(g - 2, b))
                iv, ov = in_bufs[b], out_bufs[b]

                def vec_body(i):
                    xv = iv[pl.ds(i * _L, _L)]
                    idx = xv.astype(jnp.int32)
                    ov[pl.ds(i * _L, _L)] = plsc.load_gather(full_v, [idx])

                plsc.parallel_loop(0, n_vec, 1, unroll=16)(vec_body)
                start_out(g, b)
                pl.when(g2 < n_half - 1)(lambda: start_in(g + 2, b))
            return 0

        lax.fori_loop(0, n_half, outer, 0)
        wait_out(n_chunks - 2, 0)
        wait_out(n_chunks - 1, 1)

    return k


def kernel(x, lut_embedding, n):
    shape = x.shape
    n_total = x.size
    xf = x.reshape(n_total)
    la = lut_embedding[:, 0]
    lb = lut_embedding[:, 1]
    p = jnp.exp2(jnp.asarray(n, jnp.float32) - 1.0)
    mn = jnp.full((_L,), 0.0, jnp.float32) - p
    mx = jnp.full((_L,), -1.0, jnp.float32) + p
    out = _make_sc_kernel(n_total, 6144)(xf, la, lb, mn, mx)
    return out.reshape(shape)


# speedup vs baseline: 3.3476x; 3.3476x over previous
import functools

import jax
import jax.numpy as jnp
from jax import lax
from jax.experimental import pallas as pl
from jax.experimental.pallas import tpu as pltpu
from jax.experimental.pallas import tpu_sc as plsc

_NC = 2
_NS = 16
_NW = _NC * _NS
_L = 16


@functools.lru_cache(maxsize=None)
def _make_sc_kernel(n_rows: int, n_cols: int, blk: int):
    assert n_rows % (_NW * blk) == 0
    per_w = n_rows // _NW
    n_chunks = per_w // blk
    assert n_chunks % 2 == 0
    n_half = n_chunks // 2
    n_cvec = n_cols // _L

    mesh = plsc.VectorSubcoreMesh(core_axis_name="c", subcore_axis_name="s")

    @functools.partial(
        pl.kernel,
        out_type=jax.ShapeDtypeStruct((n_rows, n_cols), jnp.float32),
        mesh=mesh,
        compiler_params=pltpu.CompilerParams(
            needs_layout_passes=False, use_tc_tiling_on_sc=True),
        scratch_types=[
            pltpu.VMEM((_L,), jnp.float32),
            pltpu.VMEM((_L,), jnp.float32),
            pltpu.VMEM((_L,), jnp.float32),
            pltpu.VMEM((_L,), jnp.float32),
            pltpu.VMEM((16 * _L,), jnp.float32),
            pltpu.VMEM((blk, n_cols), jnp.float32),
            pltpu.VMEM((blk, n_cols), jnp.float32),
            pltpu.VMEM((blk, n_cols), jnp.float32),
            pltpu.VMEM((blk, n_cols), jnp.float32),
            pltpu.SemaphoreType.DMA,
            pltpu.SemaphoreType.DMA,
            pltpu.SemaphoreType.DMA,
            pltpu.SemaphoreType.DMA,
        ],
    )
    def k(x_hbm, la_hbm, lb_hbm, mn_hbm, mx_hbm, out_hbm,
          la_v, lb_v, mn_v, mx_v, full_v,
          in_v0, in_v1, out_v0, out_v1,
          in_s0, in_s1, out_s0, out_s1):
        in_bufs, out_bufs = (in_v0, in_v1), (out_v0, out_v1)
        in_sems, out_sems = (in_s0, in_s1), (out_s0, out_s1)
        wid = lax.axis_index("s") * _NC + lax.axis_index("c")
        base = wid * per_w

        pltpu.sync_copy(la_hbm, la_v)
        pltpu.sync_copy(lb_hbm, lb_v)
        pltpu.sync_copy(mn_hbm, mn_v)
        pltpu.sync_copy(mx_hbm, mx_v)

        a_vec = la_v[...]
        b_vec = lb_v[...]
        mn = mn_v[...]
        mx = mx_v[...]
        row_ids = lax.iota(jnp.int32, _L) * _L
        for l in range(16):
            y = a_vec * (l / 16.0) + b_vec
            yf = y.astype(jnp.int32).astype(jnp.float32)
            plsc.store_scatter(full_v, [row_ids + l], jnp.clip(yf, mn, mx))

        def start_in(g, b):
            r0 = base + g * blk
            pltpu.async_copy(x_hbm.at[pl.ds(r0, blk), :], in_bufs[b], in_sems[b])

        def start_out(g, b):
            r0 = base + g * blk
            pltpu.async_copy(out_bufs[b], out_hbm.at[pl.ds(r0, blk), :], out_sems[b])

        def wait_in(g, b):
            r0 = base + g * blk
            pltpu.make_async_copy(
                x_hbm.at[pl.ds(r0, blk), :], in_bufs[b], in_sems[b]).wait()

        def wait_out(g, b):
            r0 = base + g * blk
            pltpu.make_async_copy(
                out_bufs[b], out_hbm.at[pl.ds(r0, blk), :], out_sems[b]).wait()

        start_in(0, 0)
        start_in(1, 1)

        def outer(g2, _):
            for b in range(2):
                g = g2 * 2 + b
                wait_in(g, b)
                pl.when(g2 >= 1)(lambda: wait_out(g - 2, b))
                iv, ov = in_bufs[b], out_bufs[b]

                def row_body(r):
                    for c in range(n_cvec):
                        xv = iv[r, pl.ds(c * _L, _L)]
                        idx = xv.astype(jnp.int32)
                        ov[r, pl.ds(c * _L, _L)] = plsc.load_gather(full_v, [idx])

                plsc.parallel_loop(0, blk, 1, unroll=2)(row_body)
                start_out(g, b)
                pl.when(g2 < n_half - 1)(lambda: start_in(g + 2, b))
            return 0

        lax.fori_loop(0, n_half, outer, 0)
        wait_out(n_chunks - 2, 0)
        wait_out(n_chunks - 1, 1)

    return k


def kernel(x, lut_embedding, n):
    shape = x.shape
    n_cols = shape[-1]
    n_rows = x.size // n_cols
    x2 = x.reshape(n_rows, n_cols)
    la = lut_embedding[:, 0]
    lb = lut_embedding[:, 1]
    p = jnp.exp2(jnp.asarray(n, jnp.float32) - 1.0)
    mn = jnp.full((_L,), 0.0, jnp.float32) - p
    mx = jnp.full((_L,), -1.0, jnp.float32) + p
    out = _make_sc_kernel(n_rows, n_cols, 96)(x2, la, lb, mn, mx)
    return out.reshape(shape)
